# baseline (speedup 1.0000x reference)
import functools

import jax
import jax.numpy as jnp
from jax import lax
from jax.experimental import pallas as pl
from jax.experimental.pallas import tpu as pltpu
from jax.experimental.pallas import tpu_sc as plsc

_N, _H, _S, _D = 8, 8, 32, 128
_NUM_SINK = 4
_TOTAL = 512 * 4
_REST = _S - _NUM_SINK
_P = _N * _H
_NC, _NS = 2, 16
_NW = _NC * _NS
_PPW = _P // _NW
_L = 16

_CD = _TOTAL * _D
_KD = _S * _D
_SKD = _NUM_SINK * _D
_HEADW = _REST * _D
_TAILW = _CD - _HEADW


def _vgather(x, idx):
    return lax.gather(
        x, idx[:, None],
        lax.GatherDimensionNumbers(offset_dims=(), collapsed_slice_dims=(0,),
                                   start_index_map=(0,)),
        slice_sizes=(1,), mode=lax.GatherScatterMode.PROMISE_IN_BOUNDS)


def _body(key_r, value_r, score_r, ck_in, cv_in, cs_in, op_in, mk_in,
          sk_out, sv_out, sp_out, sm_out, ck_out, cv_out, cs_out, op_out,
          mk_out, s_f, p_i, m_f, sc_b, spos, smask, sem):
    wid = lax.axis_index("s") * _NC + lax.axis_index("c")
    lane = lax.iota(jnp.int32, _L)
    fz = jnp.zeros((_L,), jnp.float32)

    big = []
    stage = []
    for p in range(_PPW):
        pr = wid * _PPW + p
        big.append(pltpu.async_copy(
            ck_in.at[pl.ds(pr * _CD + _HEADW, _TAILW)],
            ck_out.at[pl.ds(pr * _CD + _HEADW, _TAILW)], sem))
        big.append(pltpu.async_copy(
            cv_in.at[pl.ds(pr * _CD + _HEADW, _TAILW)],
            cv_out.at[pl.ds(pr * _CD + _HEADW, _TAILW)], sem))
        big.append(pltpu.async_copy(
            key_r.at[pl.ds(pr * _KD + _SKD, _HEADW)],
            ck_out.at[pl.ds(pr * _CD, _HEADW)], sem))
        big.append(pltpu.async_copy(
            value_r.at[pl.ds(pr * _KD + _SKD, _HEADW)],
            cv_out.at[pl.ds(pr * _CD, _HEADW)], sem))
        big.append(pltpu.async_copy(
            key_r.at[pl.ds(pr * _KD, _SKD)],
            sk_out.at[pl.ds(pr * _SKD, _SKD)], sem))
        big.append(pltpu.async_copy(
            value_r.at[pl.ds(pr * _KD, _SKD)],
            sv_out.at[pl.ds(pr * _SKD, _SKD)], sem))
        stage.append(pltpu.async_copy(
            cs_in.at[pl.ds(pr * _TOTAL, _TOTAL)],
            s_f.at[pl.ds(p * _TOTAL, _TOTAL)], sem))
        stage.append(pltpu.async_copy(
            op_in.at[pl.ds(pr * _TOTAL, _TOTAL)],
            p_i.at[pl.ds(p * _TOTAL, _TOTAL)], sem))
        stage.append(pltpu.async_copy(
            mk_in.at[pl.ds(pr * _TOTAL, _TOTAL)],
            m_f.at[pl.ds(p * _TOTAL, _TOTAL)], sem))
        stage.append(pltpu.async_copy(
            score_r.at[pl.ds(pr * _S, _S)],
            sc_b.at[pl.ds(p * _S, _S)], sem))
    for c in stage:
        c.wait()

    for p in range(_PPW):
        pr = wid * _PPW + p
        base = p * _TOTAL
        k12 = lane < (_REST - _L)
        a = sc_b[pl.ds(p * _S, _L)]
        b = sc_b[pl.ds(p * _S + _L, _L)]
        sh = jnp.minimum(lane + _NUM_SINK, _L - 1)
        lo = jnp.maximum(lane - (_L - _NUM_SINK), 0)
        s_f[pl.ds(base, _L)] = jnp.where(k12, _vgather(a, sh), _vgather(b, lo))
        old_s = s_f[pl.ds(base + _L, _L)]
        s_f[pl.ds(base + _L, _L)] = jnp.where(k12, _vgather(b, sh), old_s)
        p_i[pl.ds(base, _L)] = lane + _NUM_SINK
        old_p = p_i[pl.ds(base + _L, _L)]
        p_i[pl.ds(base + _L, _L)] = jnp.where(k12, lane + _L + _NUM_SINK,
                                              old_p)
        m_f[pl.ds(base, _L)] = fz
        old_m = m_f[pl.ds(base + _L, _L)]
        m_f[pl.ds(base + _L, _L)] = jnp.where(k12, fz, old_m)
        big.append(pltpu.async_copy(
            s_f.at[pl.ds(base, _TOTAL)],
            cs_out.at[pl.ds(pr * _TOTAL, _TOTAL)], sem))
        big.append(pltpu.async_copy(
            p_i.at[pl.ds(base, _TOTAL)],
            op_out.at[pl.ds(pr * _TOTAL, _TOTAL)], sem))
        big.append(pltpu.async_copy(
            m_f.at[pl.ds(base, _TOTAL)],
            mk_out.at[pl.ds(pr * _TOTAL, _TOTAL)], sem))

    @pl.when(wid == _NW - 1)
    def _():
        pat = lax.rem(lane, _NUM_SINK)
        for j in range(_P * _NUM_SINK // _L):
            spos[pl.ds(j * _L, _L)] = pat
            smask[pl.ds(j * _L, _L)] = fz
        pltpu.sync_copy(spos, sp_out)
        pltpu.sync_copy(smask, sm_out)

    for c in big:
        c.wait()


def kernel(key, value, score, sink_k, sink_v, sink_pos, sink_mask,
           cache_k, cache_v, cache_s, og_pos, mask):
    idt = og_pos.dtype
    mesh = plsc.VectorSubcoreMesh(core_axis_name="c", subcore_axis_name="s",
                                  num_cores=_NC, num_subcores=_NS)
    out_type = (
        jax.ShapeDtypeStruct((_P * _SKD,), jnp.float32),
        jax.ShapeDtypeStruct((_P * _SKD,), jnp.float32),
        jax.ShapeDtypeStruct((_P * _NUM_SINK,), idt),
        jax.ShapeDtypeStruct((_P * _NUM_SINK,), jnp.float32),
        jax.ShapeDtypeStruct((_P * _CD,), jnp.float32),
        jax.ShapeDtypeStruct((_P * _CD,), jnp.float32),
        jax.ShapeDtypeStruct((_P * _TOTAL,), jnp.float32),
        jax.ShapeDtypeStruct((_P * _TOTAL,), idt),
        jax.ShapeDtypeStruct((_P * _TOTAL,), jnp.float32),
    )
    scratch = [
        pltpu.VMEM((_PPW * _TOTAL,), jnp.float32),
        pltpu.VMEM((_PPW * _TOTAL,), idt),
        pltpu.VMEM((_PPW * _TOTAL,), jnp.float32),
        pltpu.VMEM((_PPW * _S,), jnp.float32),
        pltpu.VMEM((_P * _NUM_SINK,), idt),
        pltpu.VMEM((_P * _NUM_SINK,), jnp.float32),
        pltpu.SemaphoreType.DMA,
    ]
    run = functools.partial(pl.kernel, mesh=mesh, out_type=out_type,
                            scratch_types=scratch)(_body)
    (sk, sv, sp, sm, cko, cvo, cso, opo, mko) = run(
        key.reshape(-1), value.reshape(-1), score.reshape(-1),
        cache_k.reshape(-1), cache_v.reshape(-1), cache_s.reshape(-1),
        og_pos.reshape(-1), mask.reshape(-1))
    return (sk.reshape(_N, _H, _NUM_SINK, _D),
            sv.reshape(_N, _H, _NUM_SINK, _D),
            sp.reshape(_N, _H, _NUM_SINK),
            sm.reshape(_N, _H, _NUM_SINK),
            cko.reshape(_N, _H, _TOTAL, _D),
            cvo.reshape(_N, _H, _TOTAL, _D),
            cso.reshape(_N, _H, _TOTAL),
            opo.reshape(_N, _H, _TOTAL),
            mko.reshape(_N, _H, _TOTAL))


# speedup vs baseline: 32.5608x; 32.5608x over previous
import functools

import jax
import jax.numpy as jnp
from jax import lax
from jax.experimental import pallas as pl
from jax.experimental.pallas import tpu as pltpu
from jax.experimental.pallas import tpu_sc as plsc

_N, _H, _S, _D = 8, 8, 32, 128
_NUM_SINK = 4
_TOTAL = 512 * 4
_REST = _S - _NUM_SINK
_P = _N * _H
_NC, _NS = 2, 16
_NW = _NC * _NS
_PPW = _P // _NW
_L = 16

_CD = _TOTAL * _D
_KD = _S * _D
_SKD = _NUM_SINK * _D
_HEADW = _REST * _D
_TAILW = _CD - _HEADW

_NB = 4
_NCH = 16
_CHUNK = _TAILW // _NCH


def _vgather(x, idx):
    return lax.gather(
        x, idx[:, None],
        lax.GatherDimensionNumbers(offset_dims=(), collapsed_slice_dims=(0,),
                                   start_index_map=(0,)),
        slice_sizes=(1,), mode=lax.GatherScatterMode.PROMISE_IN_BOUNDS)


def _body(key_r, value_r, score_r, ck_in, cv_in, cs_in, op_in, mk_in,
          sk_out, sv_out, sp_out, sm_out, ck_out, cv_out, cs_out, op_out,
          mk_out, ring, s_f, p_i, m_f, sc_b, spos, smask, rsem, wsem, sem):
    wid = lax.axis_index("s") * _NC + lax.axis_index("c")
    lane = lax.iota(jnp.int32, _L)
    fz = jnp.zeros((_L,), jnp.float32)

    stage = []
    for p in range(_PPW):
        pr = wid * _PPW + p
        stage.append(pltpu.async_copy(
            cs_in.at[pl.ds(pr * _TOTAL, _TOTAL)],
            s_f.at[pl.ds(p * _TOTAL, _TOTAL)], sem))
        stage.append(pltpu.async_copy(
            op_in.at[pl.ds(pr * _TOTAL, _TOTAL)],
            p_i.at[pl.ds(p * _TOTAL, _TOTAL)], sem))
        stage.append(pltpu.async_copy(
            mk_in.at[pl.ds(pr * _TOTAL, _TOTAL)],
            m_f.at[pl.ds(p * _TOTAL, _TOTAL)], sem))
        stage.append(pltpu.async_copy(
            score_r.at[pl.ds(pr * _S, _S)],
            sc_b.at[pl.ds(p * _S, _S)], sem))

    segs = []
    for p in range(_PPW):
        pr = wid * _PPW + p
        segs.append((key_r, pr * _KD, _KD,
                     [(sk_out, pr * _SKD, 0, _SKD),
                      (ck_out, pr * _CD, _SKD, _HEADW)]))
        segs.append((value_r, pr * _KD, _KD,
                     [(sv_out, pr * _SKD, 0, _SKD),
                      (cv_out, pr * _CD, _SKD, _HEADW)]))
        for c in range(_NCH):
            off = pr * _CD + _HEADW + c * _CHUNK
            segs.append((ck_in, off, _CHUNK, [(ck_out, off, 0, _CHUNK)]))
            off = pr * _CD + _HEADW + c * _CHUNK
            segs.append((cv_in, off, _CHUNK, [(cv_out, off, 0, _CHUNK)]))

    reads = [None] * len(segs)
    writes = []

    def _write(i):
        src_ref, src_off, words, outs = segs[i]
        reads[i].wait()
        b = i % _NB
        for dst_ref, dst_off, coff, cwords in outs:
            writes.append(pltpu.async_copy(
                ring.at[pl.ds(b * _CHUNK + coff, cwords)],
                dst_ref.at[pl.ds(dst_off, cwords)], wsem))

    nw_per = [len(s[3]) for s in segs]
    drained = 0
    for i, (src_ref, src_off, words, outs) in enumerate(segs):
        b = i % _NB
        if i >= _NB:
            need = sum(nw_per[:i - _NB + 1])
            while drained < need:
                writes[drained].wait()
                drained += 1
        reads[i] = pltpu.async_copy(
            src_ref.at[pl.ds(src_off, words)],
            ring.at[pl.ds(b * _CHUNK, words)], rsem)
        if i >= 1:
            _write(i - 1)
    _write(len(segs) - 1)

    for c in stage:
        c.wait()
    for p in range(_PPW):
        pr = wid * _PPW + p
        base = p * _TOTAL
        k12 = lane < (_REST - _L)
        a = sc_b[pl.ds(p * _S, _L)]
        b = sc_b[pl.ds(p * _S + _L, _L)]
        sh = jnp.minimum(lane + _NUM_SINK, _L - 1)
        lo = jnp.maximum(lane - (_L - _NUM_SINK), 0)
        s_f[pl.ds(base, _L)] = jnp.where(k12, _vgather(a, sh), _vgather(b, lo))
        old_s = s_f[pl.ds(base + _L, _L)]
        s_f[pl.ds(base + _L, _L)] = jnp.where(k12, _vgather(b, sh), old_s)
        p_i[pl.ds(base, _L)] = lane + _NUM_SINK
        old_p = p_i[pl.ds(base + _L, _L)]
        p_i[pl.ds(base + _L, _L)] = jnp.where(k12, lane + _L + _NUM_SINK,
                                              old_p)
        m_f[pl.ds(base, _L)] = fz
        old_m = m_f[pl.ds(base + _L, _L)]
        m_f[pl.ds(base + _L, _L)] = jnp.where(k12, fz, old_m)
        writes.append(pltpu.async_copy(
            s_f.at[pl.ds(base, _TOTAL)],
            cs_out.at[pl.ds(pr * _TOTAL, _TOTAL)], wsem))
        writes.append(pltpu.async_copy(
            p_i.at[pl.ds(base, _TOTAL)],
            op_out.at[pl.ds(pr * _TOTAL, _TOTAL)], wsem))
        writes.append(pltpu.async_copy(
            m_f.at[pl.ds(base, _TOTAL)],
            mk_out.at[pl.ds(pr * _TOTAL, _TOTAL)], wsem))

    @pl.when(wid == _NW - 1)
    def _():
        pat = lax.rem(lane, _NUM_SINK)
        for j in range(_P * _NUM_SINK // _L):
            spos[pl.ds(j * _L, _L)] = pat
            smask[pl.ds(j * _L, _L)] = fz
        pltpu.sync_copy(spos, sp_out)
        pltpu.sync_copy(smask, sm_out)

    for w in writes[drained:]:
        w.wait()


def kernel(key, value, score, sink_k, sink_v, sink_pos, sink_mask,
           cache_k, cache_v, cache_s, og_pos, mask):
    idt = og_pos.dtype
    mesh = plsc.VectorSubcoreMesh(core_axis_name="c", subcore_axis_name="s",
                                  num_cores=_NC, num_subcores=_NS)
    out_type = (
        jax.ShapeDtypeStruct((_P * _SKD,), jnp.float32),
        jax.ShapeDtypeStruct((_P * _SKD,), jnp.float32),
        jax.ShapeDtypeStruct((_P * _NUM_SINK,), idt),
        jax.ShapeDtypeStruct((_P * _NUM_SINK,), jnp.float32),
        jax.ShapeDtypeStruct((_P * _CD,), jnp.float32),
        jax.ShapeDtypeStruct((_P * _CD,), jnp.float32),
        jax.ShapeDtypeStruct((_P * _TOTAL,), jnp.float32),
        jax.ShapeDtypeStruct((_P * _TOTAL,), idt),
        jax.ShapeDtypeStruct((_P * _TOTAL,), jnp.float32),
    )
    scratch = [
        pltpu.VMEM((_NB * _CHUNK,), jnp.float32),
        pltpu.VMEM((_PPW * _TOTAL,), jnp.float32),
        pltpu.VMEM((_PPW * _TOTAL,), idt),
        pltpu.VMEM((_PPW * _TOTAL,), jnp.float32),
        pltpu.VMEM((_PPW * _S,), jnp.float32),
        pltpu.VMEM((_P * _NUM_SINK,), idt),
        pltpu.VMEM((_P * _NUM_SINK,), jnp.float32),
        pltpu.SemaphoreType.DMA,
        pltpu.SemaphoreType.DMA,
        pltpu.SemaphoreType.DMA,
    ]
    run = functools.partial(pl.kernel, mesh=mesh, out_type=out_type,
                            scratch_types=scratch)(_body)
    (sk, sv, sp, sm, cko, cvo, cso, opo, mko) = run(
        key.reshape(-1), value.reshape(-1), score.reshape(-1),
        cache_k.reshape(-1), cache_v.reshape(-1), cache_s.reshape(-1),
        og_pos.reshape(-1), mask.reshape(-1))
    return (sk.reshape(_N, _H, _NUM_SINK, _D),
            sv.reshape(_N, _H, _NUM_SINK, _D),
            sp.reshape(_N, _H, _NUM_SINK),
            sm.reshape(_N, _H, _NUM_SINK),
            cko.reshape(_N, _H, _TOTAL, _D),
            cvo.reshape(_N, _H, _TOTAL, _D),
            cso.reshape(_N, _H, _TOTAL),
            opo.reshape(_N, _H, _TOTAL),
            mko.reshape(_N, _H, _TOTAL))
